# baseline (speedup 1.0000x reference)
import jax
import jax.numpy as jnp
from jax import lax
from jax.experimental import pallas as pl
from jax.experimental.pallas import tpu as pltpu
from jax.experimental.pallas import tpu_sc as plsc

VOCAB = 1000000
D = 64
B = 16384
C = 50

NC = 2
NS = 16
L = 16
NW = NC * NS
QPW = B // NW
QCHUNK = 16
NCHUNK = QPW // QCHUNK
IDX_MINOR = 80
NSTREAM = (QCHUNK * C) // IDX_MINOR
W2 = 2 * D


def _body(qsh_hbm, qpar_hbm, csh_hbm, cpar_hbm, table_hbm, out_hbm,
          cidx0, cidx1, qidx0, qidx1, rows0, rows1, qrows0, qrows1,
          scores_v, cparv0, cparv1, qparv0, qparv1, sem0, sem1):
    wid = lax.axis_index("s") * NC + lax.axis_index("c")
    lanes = lax.iota(jnp.int32, L)
    cidx = (cidx0, cidx1)
    qidx = (qidx0, qidx1)
    rows = (rows0, rows1)
    qrows = (qrows0, qrows1)
    cparv = (cparv0, cparv1)
    qparv = (qparv0, qparv1)
    sems = (sem0, sem1)

    def issue(buf, t):
        qbase = wid * QPW + t * QCHUNK
        coff = pl.multiple_of(qbase * C, 8)
        pltpu.sync_copy(csh_hbm.at[pl.ds(coff, QCHUNK * C)], cidx[buf])
        pltpu.sync_copy(cpar_hbm.at[pl.ds(coff, QCHUNK * C)],
                        cparv[buf].at[pl.ds(0, QCHUNK * C)])
        qoff = pl.multiple_of(qbase, 8)
        pltpu.sync_copy(qsh_hbm.at[pl.ds(qoff, QCHUNK)], qidx[buf])
        pltpu.sync_copy(qpar_hbm.at[pl.ds(qoff, QCHUNK)],
                        qparv[buf].at[pl.ds(0, QCHUNK)])
        for j in range(NSTREAM):
            pltpu.async_copy(
                table_hbm.at[cidx[buf].at[pl.ds(j * IDX_MINOR, IDX_MINOR)]],
                rows[buf].at[pl.ds(j * IDX_MINOR, IDX_MINOR)], sems[buf])
        pltpu.async_copy(table_hbm.at[qidx[buf]], qrows[buf], sems[buf])

    def drain(buf):
        for j in range(NSTREAM):
            pltpu.make_async_copy(
                table_hbm.at[cidx[buf].at[pl.ds(j * IDX_MINOR, IDX_MINOR)]],
                rows[buf].at[pl.ds(j * IDX_MINOR, IDX_MINOR)],
                sems[buf]).wait()
        pltpu.make_async_copy(table_hbm.at[qidx[buf]], qrows[buf],
                              sems[buf]).wait()

    def compute(buf, t):
        qbase = wid * QPW + t * QCHUNK
        rows_v = rows[buf]
        qrows_v = qrows[buf]
        cpar_v = cparv[buf]
        qpv = qparv[buf][pl.ds(0, L)]

        def q_body(q, carry2):
            qoff = jnp.sum(jnp.where(lanes == q, qpv, 0))
            qv = []
            for k in range(2):
                a, b = plsc.unpack(qrows_v[q, pl.ds(qoff + k * 32, 32)],
                                   format=plsc.PackFormat.INTERLEAVED)
                qv += [a, b]
            for c0 in range(0, C, L):
                n = min(L, C - c0)
                cpv = cpar_v[pl.ds(q * C + c0, L)]
                cur = jnp.zeros((L,), jnp.float32)
                for jj in range(n):
                    row = q * C + (c0 + jj)
                    off = cpv[jj]
                    p = None
                    for k in range(2):
                        a, b = plsc.unpack(
                            rows_v[row, pl.ds(off + k * 32, 32)],
                            format=plsc.PackFormat.INTERLEAVED)
                        pk = qv[2 * k] * a + qv[2 * k + 1] * b
                        p = pk if p is None else p + pk
                    s = jnp.sum(p)
                    cur = jnp.where(lanes == jj, s, cur)
                sig = 1.0 / (1.0 + jnp.exp(-cur))
                scores_v[pl.ds(q * C + c0, L)] = sig
            return carry2

        lax.fori_loop(0, QCHUNK, q_body, 0, unroll=False)
        pltpu.sync_copy(
            scores_v.at[pl.ds(0, QCHUNK * C)],
            out_hbm.at[pl.ds(pl.multiple_of(qbase * C, 8), QCHUNK * C)])

    issue(0, 0)

    def pair_body(tt, carry):
        t0 = 2 * tt
        drain(0)
        issue(1, t0 + 1)
        compute(0, t0)
        drain(1)

        @pl.when(tt + 1 < NCHUNK // 2)
        def _():
            issue(0, t0 + 2)

        compute(1, t0 + 1)
        return carry

    lax.fori_loop(0, NCHUNK // 2, pair_body, 0, unroll=False)


@jax.jit
def _frame(qsh, qpar, csh, cpar, table128):
    kern = pl.kernel(
        _body,
        out_type=jax.ShapeDtypeStruct((B * C,), jnp.float32),
        mesh=plsc.VectorSubcoreMesh(core_axis_name="c", subcore_axis_name="s",
                                    num_cores=NC, num_subcores=NS),
        compiler_params=pltpu.CompilerParams(needs_layout_passes=False,
                                             use_tc_tiling_on_sc=False),
        scratch_types=[
            pltpu.VMEM((QCHUNK * C,), jnp.int32),
            pltpu.VMEM((QCHUNK * C,), jnp.int32),
            pltpu.VMEM((QCHUNK,), jnp.int32),
            pltpu.VMEM((QCHUNK,), jnp.int32),
            pltpu.VMEM((QCHUNK * C, W2), jnp.bfloat16),
            pltpu.VMEM((QCHUNK * C, W2), jnp.bfloat16),
            pltpu.VMEM((QCHUNK, W2), jnp.bfloat16),
            pltpu.VMEM((QCHUNK, W2), jnp.bfloat16),
            pltpu.VMEM((QCHUNK * C + L,), jnp.float32),
            pltpu.VMEM((QCHUNK * C + L,), jnp.int32),
            pltpu.VMEM((QCHUNK * C + L,), jnp.int32),
            pltpu.VMEM((L,), jnp.int32),
            pltpu.VMEM((L,), jnp.int32),
            pltpu.SemaphoreType.DMA,
            pltpu.SemaphoreType.DMA,
        ],
    )
    return kern(qsh, qpar, csh, cpar, table128)


def kernel(query_id, candidate_hyper_ids, table):
    table128 = table.astype(jnp.bfloat16).reshape(VOCAB // 2, W2)
    cflat = candidate_hyper_ids.reshape(B * C)
    csh = cflat >> 1
    cpar = (cflat & 1) * D
    qsh = query_id >> 1
    qpar = (query_id & 1) * D
    out = _frame(qsh, qpar, csh, cpar, table128)
    return out.reshape(B, C)


# speedup vs baseline: 1.1522x; 1.1522x over previous
import jax
import jax.numpy as jnp
from jax import lax
from jax.experimental import pallas as pl
from jax.experimental.pallas import tpu as pltpu
from jax.experimental.pallas import tpu_sc as plsc

VOCAB = 1000000
D = 64
B = 16384
C = 50

NC = 2
NS = 16
L = 16
NW = NC * NS
QPW = B // NW
QCHUNK = 8
NCHUNK = QPW // QCHUNK
IDX_MINOR = 80
NSTREAM = (QCHUNK * C) // IDX_MINOR
KD = D // L
W2 = 2 * D


def _body(qsh_hbm, qpar_hbm, csh_hbm, cpar_hbm, table_hbm, out_hbm,
          cidx0, cidx1, qidx0, qidx1, rows0, rows1, qrows0, qrows1,
          scores_v, cparv0, cparv1, qparv0, qparv1, sem0, sem1):
    wid = lax.axis_index("s") * NC + lax.axis_index("c")
    lanes = lax.iota(jnp.int32, L)
    cidx = (cidx0, cidx1)
    qidx = (qidx0, qidx1)
    rows = (rows0, rows1)
    qrows = (qrows0, qrows1)
    cparv = (cparv0, cparv1)
    qparv = (qparv0, qparv1)
    sems = (sem0, sem1)

    def issue(buf, t):
        qbase = wid * QPW + t * QCHUNK
        coff = pl.multiple_of(qbase * C, 8)
        pltpu.sync_copy(csh_hbm.at[pl.ds(coff, QCHUNK * C)], cidx[buf])
        pltpu.sync_copy(cpar_hbm.at[pl.ds(coff, QCHUNK * C)],
                        cparv[buf].at[pl.ds(0, QCHUNK * C)])
        qoff = pl.multiple_of(qbase, 8)
        pltpu.sync_copy(qsh_hbm.at[pl.ds(qoff, QCHUNK)], qidx[buf])
        pltpu.sync_copy(qpar_hbm.at[pl.ds(qoff, QCHUNK)],
                        qparv[buf].at[pl.ds(0, QCHUNK)])
        for j in range(NSTREAM):
            pltpu.async_copy(
                table_hbm.at[cidx[buf].at[pl.ds(j * IDX_MINOR, IDX_MINOR)]],
                rows[buf].at[pl.ds(j * IDX_MINOR, IDX_MINOR)], sems[buf])
        pltpu.async_copy(table_hbm.at[qidx[buf]], qrows[buf], sems[buf])

    def drain(buf):
        for j in range(NSTREAM):
            pltpu.make_async_copy(
                table_hbm.at[cidx[buf].at[pl.ds(j * IDX_MINOR, IDX_MINOR)]],
                rows[buf].at[pl.ds(j * IDX_MINOR, IDX_MINOR)],
                sems[buf]).wait()
        pltpu.make_async_copy(table_hbm.at[qidx[buf]], qrows[buf],
                              sems[buf]).wait()

    def compute(buf, t):
        qbase = wid * QPW + t * QCHUNK
        rows_v = rows[buf]
        qrows_v = qrows[buf]
        cpar_v = cparv[buf]
        qpv = qparv[buf][pl.ds(0, L)]

        def q_body(q, carry2):
            qoff = jnp.sum(jnp.where(lanes == q, qpv, 0))
            qv = [qrows_v[q, pl.ds(qoff + k * L, L)] for k in range(KD)]
            for c0 in range(0, C, L):
                n = min(L, C - c0)
                cpv = cpar_v[pl.ds(q * C + c0, L)]
                cur = jnp.zeros((L,), jnp.float32)
                for jj in range(n):
                    row = q * C + (c0 + jj)
                    off = cpv[jj]
                    p = qv[0] * rows_v[row, pl.ds(off, L)]
                    for k in range(1, KD):
                        p = p + qv[k] * rows_v[row, pl.ds(off + k * L, L)]
                    s = jnp.sum(p)
                    cur = jnp.where(lanes == jj, s, cur)
                sig = 1.0 / (1.0 + jnp.exp(-cur))
                scores_v[pl.ds(q * C + c0, L)] = sig
            return carry2

        lax.fori_loop(0, QCHUNK, q_body, 0, unroll=False)
        pltpu.sync_copy(
            scores_v.at[pl.ds(0, QCHUNK * C)],
            out_hbm.at[pl.ds(pl.multiple_of(qbase * C, 8), QCHUNK * C)])

    issue(0, 0)

    def pair_body(tt, carry):
        t0 = 2 * tt
        drain(0)
        issue(1, t0 + 1)
        compute(0, t0)
        drain(1)

        @pl.when(tt + 1 < NCHUNK // 2)
        def _():
            issue(0, t0 + 2)

        compute(1, t0 + 1)
        return carry

    lax.fori_loop(0, NCHUNK // 2, pair_body, 0, unroll=False)


@jax.jit
def _frame(qsh, qpar, csh, cpar, table128):
    kern = pl.kernel(
        _body,
        out_type=jax.ShapeDtypeStruct((B * C,), jnp.float32),
        mesh=plsc.VectorSubcoreMesh(core_axis_name="c", subcore_axis_name="s",
                                    num_cores=NC, num_subcores=NS),
        compiler_params=pltpu.CompilerParams(needs_layout_passes=False,
                                             use_tc_tiling_on_sc=True),
        scratch_types=[
            pltpu.VMEM((QCHUNK * C,), jnp.int32),
            pltpu.VMEM((QCHUNK * C,), jnp.int32),
            pltpu.VMEM((QCHUNK,), jnp.int32),
            pltpu.VMEM((QCHUNK,), jnp.int32),
            pltpu.VMEM((QCHUNK * C, W2), jnp.float32),
            pltpu.VMEM((QCHUNK * C, W2), jnp.float32),
            pltpu.VMEM((QCHUNK, W2), jnp.float32),
            pltpu.VMEM((QCHUNK, W2), jnp.float32),
            pltpu.VMEM((QCHUNK * C + L,), jnp.float32),
            pltpu.VMEM((QCHUNK * C + L,), jnp.int32),
            pltpu.VMEM((QCHUNK * C + L,), jnp.int32),
            pltpu.VMEM((L,), jnp.int32),
            pltpu.VMEM((L,), jnp.int32),
            pltpu.SemaphoreType.DMA,
            pltpu.SemaphoreType.DMA,
        ],
    )
    return kern(qsh, qpar, csh, cpar, table128)


def kernel(query_id, candidate_hyper_ids, table):
    table128 = table.reshape(VOCAB // 2, W2)
    cflat = candidate_hyper_ids.reshape(B * C)
    csh = cflat >> 1
    cpar = (cflat & 1) * D
    qsh = query_id >> 1
    qpar = (query_id & 1) * D
    out = _frame(qsh, qpar, csh, cpar, table128)
    return out.reshape(B, C)
